# baseline (speedup 1.0000x reference)
import functools

import jax
import jax.numpy as jnp
from jax import lax
from jax.experimental import pallas as pl
from jax.experimental.pallas import tpu as pltpu
from jax.experimental.pallas import tpu_sc as plsc

EMBED = 64
EMBED_PAD = 128
BATCH = 16384
FIELDS = 26
FIELDS_PAD = 32
LANES = 16

NC, NS = 2, 16
NW = NC * NS
ROWS_W = BATCH // NW
BLK = 32
NBLK = ROWS_W // BLK
CHUNK = BLK * FIELDS


_mesh = plsc.VectorSubcoreMesh(
    core_axis_name="c", subcore_axis_name="s", num_cores=NC, num_subcores=NS
)


@functools.partial(
    pl.kernel,
    mesh=_mesh,
    out_type=jax.ShapeDtypeStruct((BATCH, FIELDS_PAD, EMBED_PAD), jnp.float32),
    scratch_types=[
        pltpu.VMEM((BLK, FIELDS_PAD), jnp.float32),
        pltpu.VMEM((2, CHUNK), jnp.int32),
        pltpu.VMEM((2 * CHUNK, EMBED), jnp.float32),
        pltpu.SemaphoreType.DMA,
        pltpu.SemaphoreType.DMA,
        pltpu.SemaphoreType.DMA,
        pltpu.SemaphoreType.DMA,
    ],
    compiler_params=pltpu.CompilerParams(
        use_tc_tiling_on_sc=False, needs_layout_passes=False
    ),
)
def _gather(idx_hbm, table_hbm, out_hbm, blk_v, idx_v, rows_v, g0, g1, s0, s1):
    wid = lax.axis_index("s") * NC + lax.axis_index("c")
    base = wid * ROWS_W
    gsem = (g0, g1)
    ssem = (s0, s1)

    def _flatten_step(d):
        def step(r, carry):
            idx_v[d, pl.ds(r * FIELDS, LANES)] = plsc.bitcast(
                blk_v[r, pl.ds(0, LANES)], jnp.int32
            )
            idx_v[d, pl.ds(r * FIELDS + FIELDS - LANES, LANES)] = plsc.bitcast(
                blk_v[r, pl.ds(FIELDS - LANES, LANES)], jnp.int32
            )
            return carry
        return step

    def _stage_and_fire(c):
        d = c % 2
        r0 = base + c * BLK
        pltpu.sync_copy(idx_hbm.at[pl.ds(r0, BLK)], blk_v)
        lax.fori_loop(0, BLK, _flatten_step(d), 0)
        pltpu.async_copy(
            table_hbm.at[idx_v.at[d]],
            rows_v.at[pl.ds(d * CHUNK, CHUNK)],
            gsem[d],
        )

    def _out_view(r0, k):
        return out_hbm.at[r0 + k, pl.ds(0, FIELDS), pl.ds(0, EMBED)]

    def _row_copy(d, k, r0, sem):
        return pltpu.make_async_copy(
            rows_v.at[pl.ds(d * CHUNK + k * FIELDS, FIELDS)],
            _out_view(r0, k),
            sem,
        )

    _stage_and_fire(0)
    for c in range(NBLK):
        d = c % 2
        if c + 1 < NBLK:
            _stage_and_fire(c + 1)
        pltpu.make_async_copy(
            table_hbm.at[idx_v.at[d]],
            rows_v.at[pl.ds(d * CHUNK, CHUNK)],
            gsem[d],
        ).wait()
        r0 = base + c * BLK

        def fire(k, r0):
            pltpu.async_copy(
                rows_v.at[pl.ds(d * CHUNK + k * FIELDS, FIELDS)],
                _out_view(r0, k),
                ssem[d],
            )
            return r0

        def drain(k, r0):
            _row_copy(d, k, r0, ssem[d]).wait()
            return r0

        lax.fori_loop(0, BLK, fire, r0)
        lax.fori_loop(0, BLK, drain, r0)


def kernel(x, weight):
    xf = jax.lax.bitcast_convert_type(x, jnp.float32)
    xp = jnp.pad(xf, ((0, 0), (0, FIELDS_PAD - FIELDS)))
    out3 = _gather(xp, weight)
    return out3[:, :FIELDS, :EMBED]
